# baseline (speedup 1.0000x reference)
import functools

import jax
import jax.numpy as jnp
from jax import lax
from jax.experimental import pallas as pl
from jax.experimental.pallas import tpu as pltpu
from jax.experimental.pallas import tpu_sc as plsc


def _pack_pair(x_ref, y_ref):
    xt = x_ref[:].T.astype(jnp.bfloat16)
    yt = y_ref[:].T.astype(jnp.bfloat16)
    ux = lax.bitcast_convert_type(xt, jnp.uint16).astype(jnp.uint32)
    uy = lax.bitcast_convert_type(yt, jnp.uint16).astype(jnp.uint32)
    return lax.bitcast_convert_type(ux | (uy << 16), jnp.float32)


def _repack_body(a_ref, b_ref, c_ref, d_ref, out_ref):
    out_ref[:, 0:64] = _pack_pair(a_ref, b_ref)
    out_ref[:, 64:128] = _pack_pair(c_ref, d_ref)


@functools.lru_cache(maxsize=None)
def _make_repack(V, D, Nb):
    nblk = pl.cdiv(V, 4 * Nb)
    nin = pl.cdiv(V, Nb)
    def _in_spec(k):
        return pl.BlockSpec(
            (D, Nb), lambda j, _k=k: (0, jnp.minimum(4 * j + _k, nin - 1)))
    return pl.pallas_call(
        _repack_body,
        grid=(nblk,),
        in_specs=[_in_spec(0), _in_spec(1), _in_spec(2), _in_spec(3)],
        out_specs=pl.BlockSpec((Nb, 2 * D), lambda j: (j, 0)),
        out_shape=jax.ShapeDtypeStruct((nblk * Nb, 2 * D), jnp.float32),
        compiler_params=pltpu.CompilerParams(
            dimension_semantics=("arbitrary",),
        ),
    )


@functools.lru_cache(maxsize=None)
def _make_sc_gather(V2, D2, B):
    info = plsc.get_sparse_core_info()
    NC, NS = info.num_cores, info.num_subcores
    NW = NC * NS
    assert B % NW == 0 and (B // NW) % 8 == 0
    b_per_w = B // NW
    mesh = plsc.VectorSubcoreMesh(core_axis_name="c", subcore_axis_name="s")

    @functools.partial(
        pl.kernel,
        mesh=mesh,
        out_type=jax.ShapeDtypeStruct((B, D2), jnp.float32),
        scratch_types=[
            pltpu.VMEM((b_per_w,), jnp.int32),
            pltpu.VMEM((b_per_w, D2), jnp.float32),
            pltpu.SemaphoreType.DMA,
        ],
    )
    def gather_k(table_hbm, idx_hbm, out_hbm, idx_v, rows_v, sem):
        wid = lax.axis_index("s") * NC + lax.axis_index("c")
        base = wid * b_per_w
        pltpu.sync_copy(idx_hbm.at[pl.ds(base, b_per_w)], idx_v)
        pltpu.async_copy(table_hbm.at[idx_v], rows_v, sem).wait()
        pltpu.sync_copy(rows_v, out_hbm.at[pl.ds(base, b_per_w)])

    return gather_k


def _matmul_body(wt_ref, embt_ref, out_ref):
    wt = wt_ref[:].astype(jnp.bfloat16)
    out_ref[:] = lax.dot_general(
        wt, embt_ref[:],
        (((0,), (0,)), ((), ())),
        preferred_element_type=jnp.float32,
    )


@functools.lru_cache(maxsize=None)
def _make_matmul(B, D, V, Vb):
    return pl.pallas_call(
        _matmul_body,
        grid=(pl.cdiv(V, Vb),),
        in_specs=[
            pl.BlockSpec((D, Vb), lambda i: (0, i)),
            pl.BlockSpec((D, B), lambda i: (0, 0)),
        ],
        out_specs=pl.BlockSpec((Vb, B), lambda i: (i, 0)),
        out_shape=jax.ShapeDtypeStruct((V, B), jnp.float32),
        compiler_params=pltpu.CompilerParams(
            dimension_semantics=("arbitrary",),
        ),
    )


def kernel(tokens, embed_table, head_w):
    B = tokens.shape[0]
    V, D = embed_table.shape
    tbl_t = embed_table.T
    Nb, LOG = 4096, 12
    table2 = _make_repack(V, D, Nb)(tbl_t, tbl_t, tbl_t, tbl_t)
    tokens2 = ((tokens >> (LOG + 2)) << LOG) | (tokens & (Nb - 1))
    emb2 = _make_sc_gather(table2.shape[0], 2 * D, B)(table2, tokens2)
    lane_hi = (tokens & (2 * Nb)) != 0
    bits_hi = (tokens & Nb) != 0
    half = jnp.where(lane_hi[:, None], emb2[:, D:], emb2[:, :D])
    u = lax.bitcast_convert_type(half, jnp.uint32)
    u16 = jnp.where(bits_hi[:, None], u >> 16, u & 0xFFFF).astype(jnp.uint16)
    emb = lax.bitcast_convert_type(u16, jnp.bfloat16)
    logits_t = _make_matmul(B, D, V, 2048)(head_w.T, emb.T)
    return logits_t.T


# speedup vs baseline: 1.0114x; 1.0114x over previous
import functools

import jax
import jax.numpy as jnp
from jax import lax
from jax.experimental import pallas as pl
from jax.experimental.pallas import tpu as pltpu
from jax.experimental.pallas import tpu_sc as plsc


def _pack_pair(x_ref, y_ref):
    xt = x_ref[:].T.astype(jnp.bfloat16)
    yt = y_ref[:].T.astype(jnp.bfloat16)
    ux = lax.bitcast_convert_type(xt, jnp.uint16).astype(jnp.uint32)
    uy = lax.bitcast_convert_type(yt, jnp.uint16).astype(jnp.uint32)
    return lax.bitcast_convert_type(ux | (uy << 16), jnp.float32)


def _repack_body(a_ref, b_ref, c_ref, d_ref, out_ref):
    out_ref[:, 0:64] = _pack_pair(a_ref, b_ref)
    out_ref[:, 64:128] = _pack_pair(c_ref, d_ref)


@functools.lru_cache(maxsize=None)
def _make_repack(V, D, Nb):
    nblk = pl.cdiv(V, 4 * Nb)
    nin = pl.cdiv(V, Nb)
    def _in_spec(k):
        return pl.BlockSpec(
            (D, Nb), lambda j, _k=k: (0, jnp.minimum(4 * j + _k, nin - 1)))
    return pl.pallas_call(
        _repack_body,
        grid=(nblk,),
        in_specs=[_in_spec(0), _in_spec(1), _in_spec(2), _in_spec(3)],
        out_specs=pl.BlockSpec((Nb, 2 * D), lambda j: (j, 0)),
        out_shape=jax.ShapeDtypeStruct((nblk * Nb, 2 * D), jnp.float32),
        compiler_params=pltpu.CompilerParams(
            dimension_semantics=("arbitrary",), skip_device_barrier=True,
        ),
    )


@functools.lru_cache(maxsize=None)
def _make_sc_gather(V2, D2, B):
    info = plsc.get_sparse_core_info()
    NC, NS = info.num_cores, info.num_subcores
    NW = NC * NS
    assert B % NW == 0 and (B // NW) % 8 == 0
    b_per_w = B // NW
    mesh = plsc.VectorSubcoreMesh(core_axis_name="c", subcore_axis_name="s")

    @functools.partial(
        pl.kernel,
        mesh=mesh,
        out_type=jax.ShapeDtypeStruct((B, D2), jnp.float32),
        scratch_types=[
            pltpu.VMEM((b_per_w,), jnp.int32),
            pltpu.VMEM((b_per_w, D2), jnp.float32),
            pltpu.SemaphoreType.DMA,
        ],
    )
    def gather_k(table_hbm, idx_hbm, out_hbm, idx_v, rows_v, sem):
        wid = lax.axis_index("s") * NC + lax.axis_index("c")
        base = wid * b_per_w
        pltpu.sync_copy(idx_hbm.at[pl.ds(base, b_per_w)], idx_v)
        pltpu.async_copy(table_hbm.at[idx_v], rows_v, sem).wait()
        pltpu.sync_copy(rows_v, out_hbm.at[pl.ds(base, b_per_w)])

    return gather_k


def _matmul_body(wt_ref, embt_ref, out_ref):
    wt = wt_ref[:].astype(jnp.bfloat16)
    out_ref[:] = lax.dot_general(
        wt, embt_ref[:],
        (((0,), (0,)), ((), ())),
        preferred_element_type=jnp.float32,
    )


@functools.lru_cache(maxsize=None)
def _make_matmul(B, D, V, Vb):
    return pl.pallas_call(
        _matmul_body,
        grid=(pl.cdiv(V, Vb),),
        in_specs=[
            pl.BlockSpec((D, Vb), lambda i: (0, i)),
            pl.BlockSpec((D, B), lambda i: (0, 0)),
        ],
        out_specs=pl.BlockSpec((Vb, B), lambda i: (i, 0)),
        out_shape=jax.ShapeDtypeStruct((V, B), jnp.float32),
        compiler_params=pltpu.CompilerParams(
            dimension_semantics=("arbitrary",), skip_device_barrier=True,
        ),
    )


def kernel(tokens, embed_table, head_w):
    B = tokens.shape[0]
    V, D = embed_table.shape
    tbl_t = embed_table.T
    Nb, LOG = 4096, 12
    table2 = _make_repack(V, D, Nb)(tbl_t, tbl_t, tbl_t, tbl_t)
    tokens2 = ((tokens >> (LOG + 2)) << LOG) | (tokens & (Nb - 1))
    emb2 = _make_sc_gather(table2.shape[0], 2 * D, B)(table2, tokens2)
    lane_hi = (tokens & (2 * Nb)) != 0
    bits_hi = (tokens & Nb) != 0
    half = jnp.where(lane_hi[:, None], emb2[:, D:], emb2[:, :D])
    u = lax.bitcast_convert_type(half, jnp.uint32)
    u16 = jnp.where(bits_hi[:, None], u >> 16, u & 0xFFFF).astype(jnp.uint16)
    emb = lax.bitcast_convert_type(u16, jnp.bfloat16)
    logits_t = _make_matmul(B, D, V, 4096)(head_w.T, emb.T)
    return logits_t.T
